# baseline (speedup 1.0000x reference)
import functools

import jax
import jax.numpy as jnp
from jax import lax
from jax.experimental import pallas as pl
from jax.experimental.pallas import tpu as pltpu
from jax.experimental.pallas import tpu_sc as plsc

NUM_ROWS = 8
EMBED_DIM = 128
BATCH = 16384

_info = plsc.get_sparse_core_info()
_NC, _NS = _info.num_cores, _info.num_subcores
_NW = _NC * _NS
_BPW = BATCH // _NW
_CHUNK = 128
_NCHUNK = _BPW // _CHUNK


def _make_sc_gather():
    mesh = plsc.VectorSubcoreMesh(core_axis_name="c", subcore_axis_name="s")

    @functools.partial(
        pl.kernel,
        mesh=mesh,
        out_type=jax.ShapeDtypeStruct((_NW, _NCHUNK, _CHUNK, EMBED_DIM),
                                      jnp.float32),
        scratch_types=[
            pltpu.VMEM((_NCHUNK, _CHUNK), jnp.int32),
            pltpu.VMEM((_NCHUNK, _CHUNK, EMBED_DIM), jnp.float32),
            pltpu.SemaphoreType.DMA,
        ],
    )
    def gather_kernel(idx_hbm, table_hbm, out_hbm, idx_v, rows_v, sem):
        wid = lax.axis_index("s") * _NC + lax.axis_index("c")
        pltpu.sync_copy(idx_hbm.at[wid], idx_v)
        copies = []
        for j in range(_NCHUNK):
            copies.append(
                pltpu.async_copy(table_hbm.at[idx_v.at[j]], rows_v.at[j], sem))
        for c in copies:
            c.wait()
        pltpu.sync_copy(rows_v, out_hbm.at[wid])

    return gather_kernel


_sc_gather = _make_sc_gather()


def kernel(indices, table):
    idx = indices.astype(jnp.int32).reshape(_NW, _NCHUNK, _CHUNK)
    out = _sc_gather(idx, table)
    return out.reshape(BATCH, EMBED_DIM)


# speedup vs baseline: 1.0132x; 1.0132x over previous
import functools

import jax
import jax.numpy as jnp
from jax import lax
from jax.experimental import pallas as pl
from jax.experimental.pallas import tpu as pltpu
from jax.experimental.pallas import tpu_sc as plsc

NUM_ROWS = 8
EMBED_DIM = 128
BATCH = 16384

_info = plsc.get_sparse_core_info()
_NC, _NS, _L = _info.num_cores, _info.num_subcores, _info.num_lanes
_NW = _NC * _NS
_BPW = BATCH // _NW
_NGROUP = _BPW // _L


def _make_sc_lookup():
    mesh = plsc.VectorSubcoreMesh(core_axis_name="c", subcore_axis_name="s")

    @functools.partial(
        pl.kernel,
        mesh=mesh,
        out_type=jax.ShapeDtypeStruct((_NW, _BPW * EMBED_DIM), jnp.float32),
        scratch_types=[
            pltpu.VMEM((NUM_ROWS * EMBED_DIM,), jnp.float32),
            pltpu.VMEM((_BPW,), jnp.int32),
            pltpu.VMEM((_BPW * EMBED_DIM,), jnp.float32),
            pltpu.SemaphoreType.DMA,
        ],
        compiler_params=pltpu.CompilerParams(needs_layout_passes=False),
    )
    def lookup_kernel(idx_hbm, table_hbm, out_hbm, table_v, idx_v, out_v, sem):
        wid = lax.axis_index("s") * _NC + lax.axis_index("c")
        pltpu.sync_copy(table_hbm, table_v)
        pltpu.sync_copy(idx_hbm.at[wid], idx_v)

        lane_off = lax.iota(jnp.int32, _L) * EMBED_DIM

        def group_body(g, carry):
            rows = idx_v[pl.ds(g * _L, _L)]
            src_base = rows * EMBED_DIM
            dst_base = g * (_L * EMBED_DIM) + lane_off
            for c in range(EMBED_DIM):
                vals = plsc.load_gather(table_v, [src_base + c])
                plsc.store_scatter(out_v, [dst_base + c], vals)
            return carry

        lax.fori_loop(0, _NGROUP, group_body, 0)
        pltpu.sync_copy(out_v, out_hbm.at[wid])

    return lookup_kernel


_sc_lookup = _make_sc_lookup()


def kernel(indices, table):
    idx = indices.astype(jnp.int32).reshape(_NW, _BPW)
    out = _sc_lookup(idx, table.reshape(NUM_ROWS * EMBED_DIM))
    return out.reshape(BATCH, EMBED_DIM)


# speedup vs baseline: 1.2713x; 1.2548x over previous
import functools

import jax
import jax.numpy as jnp
from jax import lax
from jax.experimental import pallas as pl
from jax.experimental.pallas import tpu as pltpu
from jax.experimental.pallas import tpu_sc as plsc

NUM_ROWS = 8
EMBED_DIM = 128
BATCH = 16384

_info = plsc.get_sparse_core_info()
_NC, _NS, _L = _info.num_cores, _info.num_subcores, _info.num_lanes
_NW = _NC * _NS
_BPW = BATCH // _NW
_NGROUP = _BPW // _L


def _make_sc_lookup():
    mesh = plsc.VectorSubcoreMesh(core_axis_name="c", subcore_axis_name="s")

    @functools.partial(
        pl.kernel,
        mesh=mesh,
        out_type=jax.ShapeDtypeStruct((_NW, _BPW * EMBED_DIM), jnp.float32),
        scratch_types=[
            pltpu.VMEM((NUM_ROWS * EMBED_DIM,), jnp.float32),
            pltpu.VMEM((_BPW,), jnp.int32),
            pltpu.VMEM((_BPW * EMBED_DIM,), jnp.float32),
            pltpu.SemaphoreType.DMA,
        ],
        compiler_params=pltpu.CompilerParams(needs_layout_passes=False),
    )
    def lookup_kernel(idx_hbm, table_hbm, out_hbm, table_v, idx_v, out_v, sem):
        wid = lax.axis_index("s") * _NC + lax.axis_index("c")
        pltpu.sync_copy(table_hbm, table_v)
        pltpu.sync_copy(idx_hbm.at[wid], idx_v)

        lane_off = lax.iota(jnp.int32, _L) * EMBED_DIM

        @plsc.parallel_loop(0, _NGROUP, unroll=2)
        def group_body(g):
            rows = idx_v[pl.ds(g * _L, _L)]
            src_base = rows * EMBED_DIM
            dst_base = g * (_L * EMBED_DIM) + lane_off
            for c in range(EMBED_DIM):
                vals = plsc.load_gather(table_v, [src_base + c])
                plsc.store_scatter(out_v, [dst_base + c], vals)
        pltpu.sync_copy(out_v, out_hbm.at[wid])

    return lookup_kernel


_sc_lookup = _make_sc_lookup()


def kernel(indices, table):
    idx = indices.astype(jnp.int32).reshape(_NW, _BPW)
    out = _sc_lookup(idx, table.reshape(NUM_ROWS * EMBED_DIM))
    return out.reshape(BATCH, EMBED_DIM)
